# baseline (speedup 1.0000x reference)
import functools
import math

import jax
import jax.numpy as jnp
import numpy as np
from jax import lax
from jax.experimental import pallas as pl
from jax.experimental.pallas import tpu as pltpu
from jax.experimental.pallas import tpu_sc as plsc

_VOCAB_TITLES = 1000000
_GENRE_VOCAB = 51
_EMBED = 32
_BATCH = 16384
_N_GENRES = 5
_ADAPT = np.array([1.0, 1.5, 2.0, 2.5, 3.0, 3.5, 4.0, 4.5, 5.0], dtype=np.float32)
_NORM_MEAN = float(_ADAPT.mean())
_INV_STD = float(1.0 / math.sqrt(float(_ADAPT.var())))

_OUT_W = 2 * _EMBED + 1

_info = plsc.get_sparse_core_info()
_NC, _NS, _L = _info.num_cores, _info.num_subcores, _info.num_lanes
_NW = _NC * _NS
_BW = _BATCH // _NW


def _sc_body(title_hbm, gidx_hbm, rating_hbm, ttab_hbm, gtab_hbm, out_hbm,
             idx_v, trows_v, gtab_v, gidx_v, rate_v, out_v, sem):
    wid = lax.axis_index("s") * _NC + lax.axis_index("c")
    base = wid * _BW

    pltpu.sync_copy(title_hbm.at[pl.ds(base, _BW)], idx_v)
    title_dma = pltpu.async_copy(ttab_hbm.at[idx_v], trows_v, sem)

    pltpu.sync_copy(gtab_hbm, gtab_v)
    pltpu.sync_copy(gidx_hbm.at[pl.ds(base * _N_GENRES, _BW * _N_GENRES)],
                    gidx_v.at[pl.ds(0, _BW * _N_GENRES)])
    pltpu.sync_copy(rating_hbm.at[pl.ds(base, _BW)], rate_v.at[pl.ds(0, _BW)])

    lanes = lax.iota(jnp.int32, _L)

    def genre_body(b, carry):
        gids = gidx_v[pl.ds(b * _N_GENRES, _L)]
        g0 = jnp.zeros((_L,), jnp.float32)
        g1 = jnp.zeros((_L,), jnp.float32)
        for k in range(_N_GENRES):
            gid = gids[k]
            g0 = g0 + gtab_v[gid, pl.ds(0, _L)]
            g1 = g1 + gtab_v[gid, pl.ds(_L, _L)]
        out_v[pl.ds(b * _OUT_W + _EMBED, _L)] = g0 * (1.0 / _N_GENRES)
        out_v[pl.ds(b * _OUT_W + _EMBED + _L, _L)] = g1 * (1.0 / _N_GENRES)
        return carry

    lax.fori_loop(0, _BW, genre_body, 0)

    title_dma.wait()

    def title_body(b, carry):
        out_v[pl.ds(b * _OUT_W, _L)] = trows_v[b, pl.ds(0, _L)]
        out_v[pl.ds(b * _OUT_W + _L, _L)] = trows_v[b, pl.ds(_L, _L)]
        r0 = rate_v[pl.ds(b, _L)][0]
        rn = (r0 - _NORM_MEAN) * _INV_STD
        w = out_v[pl.ds(b * _OUT_W + _OUT_W - _L, _L)]
        out_v[pl.ds(b * _OUT_W + _OUT_W - _L, _L)] = jnp.where(
            lanes == _L - 1, rn, w)
        return carry

    lax.fori_loop(0, _BW, title_body, 0)

    pltpu.sync_copy(out_v, out_hbm.at[pl.ds(base * _OUT_W, _BW * _OUT_W)])


def kernel(title, book_genres, bucketized_average_rating, title_table, genre_table):
    gidx_flat = book_genres.reshape(-1)
    mesh = plsc.VectorSubcoreMesh(core_axis_name="c", subcore_axis_name="s")
    run = functools.partial(
        pl.kernel,
        mesh=mesh,
        compiler_params=pltpu.CompilerParams(use_tc_tiling_on_sc=False),
        out_type=jax.ShapeDtypeStruct((_BATCH * _OUT_W,), jnp.float32),
        scratch_types=[
            pltpu.VMEM((_BW,), jnp.int32),
            pltpu.VMEM((_BW, _EMBED), jnp.float32),
            pltpu.VMEM((_GENRE_VOCAB, _EMBED), jnp.float32),
            pltpu.VMEM((_BW * _N_GENRES + _L,), jnp.int32),
            pltpu.VMEM((_BW + _L,), jnp.float32),
            pltpu.VMEM((_BW * _OUT_W,), jnp.float32),
            pltpu.SemaphoreType.DMA,
        ],
    )(_sc_body)
    out_flat = run(title, gidx_flat, bucketized_average_rating, title_table,
                   genre_table)
    return out_flat.reshape(_BATCH, _OUT_W)
